# baseline (speedup 1.0000x reference)
import functools

import jax
import jax.numpy as jnp
from jax import lax
from jax.experimental import pallas as pl
from jax.experimental.pallas import tpu as pltpu
from jax.experimental.pallas import tpu_sc as plsc

_H = 64
_LANES = 16
_TCHUNK = 128
_GSUB = 16
_SPAD = 17


@functools.lru_cache(maxsize=None)
def _build(n_tokens, n_pos, n_ags):
  info = plsc.get_sparse_core_info()
  nw = info.num_cores * info.num_subcores
  per_w = n_tokens // nw
  n_chunks = per_w // _TCHUNK
  mesh = plsc.VectorSubcoreMesh(core_axis_name="c", subcore_axis_name="s")

  idx_set = lambda: [pltpu.VMEM((_TCHUNK,), jnp.int32) for _ in range(6)]

  @functools.partial(
      pl.kernel,
      mesh=mesh,
      compiler_params=pltpu.CompilerParams(use_tc_tiling_on_sc=False,
                                           needs_layout_passes=False),
      out_type=jax.ShapeDtypeStruct((n_tokens, _H), jnp.float32),
      scratch_types=[
          pltpu.VMEM((_H, n_pos), jnp.float32),
          pltpu.VMEM((_H, n_ags), jnp.float32),
          pltpu.VMEM((_H,), jnp.float32),
          pltpu.VMEM((_H,), jnp.float32),
          pltpu.VMEM((_H, _SPAD), jnp.float32),
          [pltpu.VMEM((_TCHUNK, _H), jnp.float32) for _ in range(2)],
          [pltpu.VMEM((_TCHUNK, _H), jnp.float32) for _ in range(2)],
          [idx_set() for _ in range(2)],
          [pltpu.SemaphoreType.DMA for _ in range(2)],
          [pltpu.SemaphoreType.DMA for _ in range(2)],
          [pltpu.SemaphoreType.DMA for _ in range(2)],
      ],
  )
  def emb_ln(wid_h, pid_h, aid_h, gid_h, sid_h,
             wtab_h, ptab_t_h, agstab_t_h, gam_h, bet_h,
             out_h,
             ptab, agstab, gam, bet, sst,
             rows, obuf, idxs, sem_g, sem_i, sem_o):
    w = lax.axis_index("s") * info.num_cores + lax.axis_index("c")
    base_w = w * per_w

    pltpu.sync_copy(ptab_t_h, ptab)
    pltpu.sync_copy(agstab_t_h, agstab)
    pltpu.sync_copy(gam_h, gam)
    pltpu.sync_copy(bet_h, bet)

    lane = lax.iota(jnp.int32, _LANES)
    perms = [lax.bitwise_xor(lane, jnp.int32(1 << p)) for p in range(4)]
    gdn = lax.GatherDimensionNumbers(
        offset_dims=(), collapsed_slice_dims=(0,), start_index_map=(0,))

    def allsum(v):
      for p in perms:
        v = v + lax.gather(v, p[:, None], gdn, (1,),
                           mode=lax.GatherScatterMode.PROMISE_IN_BOUNDS)
      return v

    gmk = []
    btk = []
    hvk = []
    for k in range(4):
      sl = pl.ds(k * _LANES, _LANES)
      gmk.append(gam[sl])
      btk.append(bet[sl])
      hvk.append(lane + jnp.int32(k * _LANES))

    def idx_copies(c, s):
      tb = base_w + c * _TCHUNK
      widx, pidx, aidx, gidx, sidx, agsidx = idxs[s]
      return [pltpu.make_async_copy(
          src.at[pl.ds(tb, _TCHUNK)], dst, sem_i[s])
              for src, dst in ((wid_h, widx), (pid_h, pidx),
                               (aid_h, aidx), (gid_h, gidx),
                               (sid_h, sidx))]

    def ags_combine(s):
      widx, pidx, aidx, gidx, sidx, agsidx = idxs[s]

      def gbody(g, carry):
        sl = pl.ds(g * _LANES, _LANES)
        agsidx[sl] = 4 * aidx[sl] + 2 * gidx[sl] + sidx[sl]
        return carry

      lax.fori_loop(0, _TCHUNK // _LANES, gbody, 0)

    def gather_copies(c, s):
      widx = idxs[s][0]
      return [pltpu.make_async_copy(
          wtab_h.at[widx.at[pl.ds(j * _GSUB, _GSUB)]],
          rows[s].at[pl.ds(j * _GSUB, _GSUB)], sem_g[s])
              for j in range(_TCHUNK // _GSUB)]

    def out_copy(c, s):
      tb = base_w + c * _TCHUNK
      return pltpu.make_async_copy(
          obuf[s], out_h.at[pl.ds(tb, _TCHUNK)], sem_o[s])

    def compute(s):
      _, pidx, _, _, _, agsidx = idxs[s]
      rbuf = rows[s]
      wbuf = obuf[s]

      def gbody(g, carry):
        gb = g * _LANES
        pv = pidx[pl.ds(gb, _LANES)]
        agsv = agsidx[pl.ds(gb, _LANES)]
        for h in range(_H):
          hv = jnp.full((_LANES,), h, jnp.int32)
          sm = (plsc.load_gather(ptab, [hv, pv])
                + plsc.load_gather(agstab, [hv, agsv]))
          sst[h, pl.ds(0, _LANES)] = sm
        for u in range(_LANES):
          t = gb + u
          uv = jnp.full((_LANES,), u, jnp.int32)
          acc = []
          for k in range(4):
            sl = pl.ds(k * _LANES, _LANES)
            acc.append(rbuf[t, sl] + plsc.load_gather(sst, [hvk[k], uv]))
          s1 = (acc[0] + acc[1]) + (acc[2] + acc[3])
          s2 = (acc[0] * acc[0] + acc[1] * acc[1]) + (
              acc[2] * acc[2] + acc[3] * acc[3])
          tot = allsum(s1)
          tot2 = allsum(s2)
          mean = tot * (1.0 / _H)
          var = tot2 * (1.0 / _H) - mean * mean
          x = var + 1e-12
          xi = lax.bitcast_convert_type(x, jnp.int32)
          y = lax.bitcast_convert_type(
              jnp.int32(0x5F3759DF) - jnp.right_shift(xi, 1), jnp.float32)
          xh = x * 0.5
          y = y * (1.5 - xh * y * y)
          y = y * (1.5 - xh * y * y)
          ms = mean * y
          for k in range(4):
            sl = pl.ds(k * _LANES, _LANES)
            wbuf[t, sl] = (acc[k] * y - ms) * gmk[k] + btk[k]
        return carry

      lax.fori_loop(0, _TCHUNK // _LANES, gbody, 0)

    def do_chunk(c, s):
      ns = 1 - s

      @pl.when(c + 1 < n_chunks)
      def _():
        for cp in idx_copies(c + 1, ns):
          cp.wait()
        ags_combine(ns)
        for cp in gather_copies(c + 1, ns):
          cp.start()

      @pl.when(c >= 2)
      def _():
        out_copy(c - 2, s).wait()

      for cp in gather_copies(c, s):
        cp.wait()
      compute(s)

      @pl.when(c + 2 < n_chunks)
      def _():
        for cp in idx_copies(c + 2, s):
          cp.start()

      out_copy(c, s).start()

    for cp in idx_copies(0, 0):
      cp.start()
      cp.wait()
    ags_combine(0)
    for cp in gather_copies(0, 0):
      cp.start()
    for cp in idx_copies(1, 1):
      cp.start()

    def pair_body(c2, carry):
      do_chunk(2 * c2, 0)
      do_chunk(2 * c2 + 1, 1)
      return carry

    lax.fori_loop(0, n_chunks // 2, pair_body, 0)
    out_copy(n_chunks - 2, 0).wait()
    out_copy(n_chunks - 1, 1).wait()

  return emb_ln


def kernel(word_ids, seg_ids, posi_ids, age_ids, gender_ids,
           word_table, seg_table, age_table, gender_table, posi_table,
           gamma, beta):
  b, s = word_ids.shape
  n = b * s
  wi = word_ids.reshape(n).astype(jnp.int32)
  si = seg_ids.reshape(n).astype(jnp.int32)
  pi = posi_ids.reshape(n).astype(jnp.int32)
  ai = age_ids.reshape(n).astype(jnp.int32)
  gi = gender_ids.reshape(n).astype(jnp.int32)
  gs = (gender_table[:, None, :] + seg_table[None, :, :]).reshape(4, _H)
  ags = (age_table[:, None, :] + gs[None, :, :]).reshape(-1, _H)
  fn = _build(n, posi_table.shape[0], ags.shape[0])
  out = fn(wi, pi, ai, gi, si,
           word_table, posi_table.T, ags.T,
           gamma.astype(jnp.float32), beta.astype(jnp.float32))
  return out.reshape(b, s, _H)


# speedup vs baseline: 1.3877x; 1.3877x over previous
import functools

import jax
import jax.numpy as jnp
from jax import lax
from jax.experimental import pallas as pl
from jax.experimental.pallas import tpu as pltpu
from jax.experimental.pallas import tpu_sc as plsc

_H = 64
_LANES = 16
_TCHUNK = 128
_GSUB = 16
_SPAD = 17


@functools.lru_cache(maxsize=None)
def _build(n_tokens, n_pos, n_ags):
  info = plsc.get_sparse_core_info()
  nw = info.num_cores * info.num_subcores
  per_w = n_tokens // nw
  n_chunks = per_w // _TCHUNK
  mesh = plsc.VectorSubcoreMesh(core_axis_name="c", subcore_axis_name="s")

  idx_set = lambda: [pltpu.VMEM((_TCHUNK,), jnp.int32) for _ in range(6)]

  @functools.partial(
      pl.kernel,
      mesh=mesh,
      compiler_params=pltpu.CompilerParams(use_tc_tiling_on_sc=False),
      out_type=jax.ShapeDtypeStruct((n_tokens, _H), jnp.float32),
      scratch_types=[
          pltpu.VMEM((n_pos, _H), jnp.float32),
          pltpu.VMEM((n_ags, _H), jnp.float32),
          pltpu.VMEM((_H,), jnp.float32),
          pltpu.VMEM((_H,), jnp.float32),
          [pltpu.VMEM((_TCHUNK, _H), jnp.float32) for _ in range(2)],
          [pltpu.VMEM((_TCHUNK, _H), jnp.float32) for _ in range(2)],
          [idx_set() for _ in range(2)],
          [pltpu.SemaphoreType.DMA for _ in range(2)],
          [pltpu.SemaphoreType.DMA for _ in range(2)],
          [pltpu.SemaphoreType.DMA for _ in range(2)],
      ],
  )
  def emb_ln(wid_h, pid_h, aid_h, gid_h, sid_h,
             wtab_h, ptab_t_h, agstab_t_h, gam_h, bet_h,
             out_h,
             ptab, agstab, gam, bet,
             rows, obuf, idxs, sem_g, sem_i, sem_o):
    w = lax.axis_index("s") * info.num_cores + lax.axis_index("c")
    base_w = w * per_w

    pltpu.sync_copy(ptab_t_h, ptab)
    pltpu.sync_copy(agstab_t_h, agstab)
    pltpu.sync_copy(gam_h, gam)
    pltpu.sync_copy(bet_h, bet)

    lane = lax.iota(jnp.int32, _LANES)
    perms = [lax.bitwise_xor(lane, jnp.int32(1 << p)) for p in range(4)]
    gdn = lax.GatherDimensionNumbers(
        offset_dims=(), collapsed_slice_dims=(0,), start_index_map=(0,))

    def allsum(v):
      for p in perms:
        v = v + lax.gather(v, p[:, None], gdn, (1,),
                           mode=lax.GatherScatterMode.PROMISE_IN_BOUNDS)
      return v

    gmk = []
    btk = []
    for k in range(4):
      sl = pl.ds(k * _LANES, _LANES)
      gmk.append(gam[sl])
      btk.append(bet[sl])

    def idx_copies(c, s):
      tb = base_w + c * _TCHUNK
      widx, pidx, aidx, gidx, sidx, agsidx = idxs[s]
      return [pltpu.make_async_copy(
          src.at[pl.ds(tb, _TCHUNK)], dst, sem_i[s])
              for src, dst in ((wid_h, widx), (pid_h, pidx),
                               (aid_h, aidx), (gid_h, gidx),
                               (sid_h, sidx))]

    def ags_combine(s):
      widx, pidx, aidx, gidx, sidx, agsidx = idxs[s]

      def gbody(g, carry):
        sl = pl.ds(g * _LANES, _LANES)
        agsidx[sl] = 4 * aidx[sl] + 2 * gidx[sl] + sidx[sl]
        return carry

      lax.fori_loop(0, _TCHUNK // _LANES, gbody, 0)

    def gather_copies(c, s):
      widx = idxs[s][0]
      return [pltpu.make_async_copy(
          wtab_h.at[widx.at[pl.ds(j * _GSUB, _GSUB)]],
          rows[s].at[pl.ds(j * _GSUB, _GSUB)], sem_g[s])
              for j in range(_TCHUNK // _GSUB)]

    def out_copy(c, s):
      tb = base_w + c * _TCHUNK
      return pltpu.make_async_copy(
          obuf[s], out_h.at[pl.ds(tb, _TCHUNK)], sem_o[s])

    def compute(s):
      _, pidx, _, _, _, agsidx = idxs[s]
      rbuf = rows[s]
      wbuf = obuf[s]

      def gbody(g, carry):
        gb = g * _LANES
        pv = pidx[pl.ds(gb, _LANES)]
        agsv = agsidx[pl.ds(gb, _LANES)]
        pts = [pv[u] for u in range(_LANES)]
        agsts = [agsv[u] for u in range(_LANES)]
        for u in range(_LANES):
          t = gb + u
          pt = pts[u]
          agst = agsts[u]
          acc = []
          for k in range(4):
            sl = pl.ds(k * _LANES, _LANES)
            acc.append(rbuf[t, sl] + ptab[pt, sl] + agstab[agst, sl])
          s1 = (acc[0] + acc[1]) + (acc[2] + acc[3])
          s2 = (acc[0] * acc[0] + acc[1] * acc[1]) + (
              acc[2] * acc[2] + acc[3] * acc[3])
          tot = allsum(s1)
          tot2 = allsum(s2)
          mean = tot * (1.0 / _H)
          var = tot2 * (1.0 / _H) - mean * mean
          x = var + 1e-12
          xi = lax.bitcast_convert_type(x, jnp.int32)
          y = lax.bitcast_convert_type(
              jnp.int32(0x5F3759DF) - jnp.right_shift(xi, 1), jnp.float32)
          xh = x * 0.5
          y = y * (1.5 - xh * y * y)
          y = y * (1.5 - xh * y * y)
          ms = mean * y
          for k in range(4):
            sl = pl.ds(k * _LANES, _LANES)
            wbuf[t, sl] = (acc[k] * y - ms) * gmk[k] + btk[k]
        return carry

      lax.fori_loop(0, _TCHUNK // _LANES, gbody, 0)

    def do_chunk(c, s):
      ns = 1 - s

      @pl.when(c + 1 < n_chunks)
      def _():
        for cp in idx_copies(c + 1, ns):
          cp.wait()
        ags_combine(ns)
        for cp in gather_copies(c + 1, ns):
          cp.start()

      @pl.when(c >= 2)
      def _():
        out_copy(c - 2, s).wait()

      for cp in gather_copies(c, s):
        cp.wait()
      compute(s)

      @pl.when(c + 2 < n_chunks)
      def _():
        for cp in idx_copies(c + 2, s):
          cp.start()

      out_copy(c, s).start()

    for cp in idx_copies(0, 0):
      cp.start()
      cp.wait()
    ags_combine(0)
    for cp in gather_copies(0, 0):
      cp.start()
    for cp in idx_copies(1, 1):
      cp.start()

    def pair_body(c2, carry):
      do_chunk(2 * c2, 0)
      do_chunk(2 * c2 + 1, 1)
      return carry

    lax.fori_loop(0, n_chunks // 2, pair_body, 0)
    out_copy(n_chunks - 2, 0).wait()
    out_copy(n_chunks - 1, 1).wait()

  return emb_ln


def kernel(word_ids, seg_ids, posi_ids, age_ids, gender_ids,
           word_table, seg_table, age_table, gender_table, posi_table,
           gamma, beta):
  b, s = word_ids.shape
  n = b * s
  wi = word_ids.reshape(n).astype(jnp.int32)
  si = seg_ids.reshape(n).astype(jnp.int32)
  pi = posi_ids.reshape(n).astype(jnp.int32)
  ai = age_ids.reshape(n).astype(jnp.int32)
  gi = gender_ids.reshape(n).astype(jnp.int32)
  gs = (gender_table[:, None, :] + seg_table[None, :, :]).reshape(4, _H)
  ags = (age_table[:, None, :] + gs[None, :, :]).reshape(-1, _H)
  fn = _build(n, posi_table.shape[0], ags.shape[0])
  out = fn(wi, pi, ai, gi, si,
           word_table, posi_table, ags,
           gamma.astype(jnp.float32), beta.astype(jnp.float32))
  return out.reshape(b, s, _H)
